# baseline (speedup 1.0000x reference)
import functools

import jax
import jax.numpy as jnp
from jax import lax
from jax.experimental import pallas as pl
from jax.experimental.pallas import tpu as pltpu
from jax.experimental.pallas import tpu_sc as plsc

NC = 2
NS = 16
NW = NC * NS
LANES = 16


def _mm_body(x_ref, w_ref, o_ref):
    o_ref[...] = jnp.dot(x_ref[...], w_ref[...], preferred_element_type=jnp.float32)


def _matmul(X, W):
    n, d_in = X.shape
    d_out = W.shape[1]
    blk = 2000
    grid = n // blk
    return pl.pallas_call(
        _mm_body,
        grid=(grid,),
        in_specs=[
            pl.BlockSpec((blk, d_in), lambda i: (i, 0)),
            pl.BlockSpec((d_in, d_out), lambda i: (0, 0)),
        ],
        out_specs=pl.BlockSpec((blk, d_out), lambda i: (i, 0)),
        out_shape=jax.ShapeDtypeStruct((n, d_out), jnp.float32),
    )(X, W)


def _combine_body(p_ref, b_ref, o_ref):
    o_ref[...] = p_ref[0] + p_ref[1] + b_ref[...]


def _combine(partial, b):
    _, n, d = partial.shape
    blk = 2000
    grid = n // blk
    return pl.pallas_call(
        _combine_body,
        grid=(grid,),
        in_specs=[
            pl.BlockSpec((2, blk, d), lambda i: (0, i, 0)),
            pl.BlockSpec((1, d), lambda i: (0, 0)),
        ],
        out_specs=pl.BlockSpec((blk, d), lambda i: (i, 0)),
        out_shape=jax.ShapeDtypeStruct((n, d), jnp.float32),
    )(partial, b.reshape(1, d))


K = 128
NROW = 2
NIDX = 4


@jax.jit
def _sc_spmm(xw, srcm, dstm, ewm, zeros):
    _, ch, _ = srcm.shape
    d = xw.shape[1]
    n_pad = zeros.shape[0]
    rows_per_tile = n_pad // NS
    jcount = d // LANES
    mesh = plsc.VectorSubcoreMesh(core_axis_name="c", subcore_axis_name="s")

    @functools.partial(
        pl.kernel,
        out_type=jax.ShapeDtypeStruct((NC, n_pad, d), jnp.float32),
        mesh=mesh,
        scratch_types=[
            pltpu.VMEM((NIDX, K), jnp.int32),
            pltpu.VMEM((NIDX, K), jnp.int32),
            pltpu.VMEM((NIDX, K), jnp.float32),
            pltpu.VMEM((NROW, K, d), jnp.float32),
            pltpu.VMEM_SHARED((n_pad, d), jnp.float32),
            pltpu.SemaphoreType.DMA((NROW,)),
            pltpu.SemaphoreType.DMA((NIDX,)),
        ],
    )
    def spmm(xw_hbm, srcm_hbm, dstm_hbm, ewm_hbm, z_hbm, out_hbm,
             src_r, dst_r, ew_r, rows, acc, gsem, isem):
        c = lax.axis_index("c")
        s = lax.axis_index("s")
        wid = s * NC + c

        r0 = s * rows_per_tile
        pltpu.sync_copy(z_hbm.at[pl.ds(r0, rows_per_tile)],
                        acc.at[pl.ds(r0, rows_per_tile)])
        plsc.subcore_barrier()

        dnums = lax.GatherDimensionNumbers(
            offset_dims=(), collapsed_slice_dims=(0,), start_index_map=(0,))

        def istart(ci, q):
            pltpu.async_copy(srcm_hbm.at[wid, ci], src_r.at[q], isem.at[q])
            pltpu.async_copy(dstm_hbm.at[wid, ci], dst_r.at[q], isem.at[q])
            pltpu.async_copy(ewm_hbm.at[wid, ci], ew_r.at[q], isem.at[q])

        def iwait(ci, q):
            pltpu.make_async_copy(
                srcm_hbm.at[wid, ci], src_r.at[q], isem.at[q]).wait()
            pltpu.make_async_copy(
                dstm_hbm.at[wid, ci], dst_r.at[q], isem.at[q]).wait()
            pltpu.make_async_copy(
                ewm_hbm.at[wid, ci], ew_r.at[q], isem.at[q]).wait()

        def gstart(q, b):
            pltpu.async_copy(xw_hbm.at[src_r.at[q]], rows.at[b], gsem.at[b])

        def gwait(q, b):
            pltpu.make_async_copy(
                xw_hbm.at[src_r.at[q]], rows.at[b], gsem.at[b]).wait()

        def scale(q, b):
            def group_body(g, rc):
                wv = ew_r[q, pl.ds(g * LANES, LANES)]
                for r in range(LANES):
                    w = lax.gather(
                        wv, jnp.full((LANES, 1), r, jnp.int32), dnums,
                        slice_sizes=(1,),
                        mode=lax.GatherScatterMode.PROMISE_IN_BOUNDS)
                    e = g * LANES + r
                    for j in range(jcount):
                        sl = pl.ds(j * LANES, LANES)
                        rows[b, e, sl] = rows[b, e, sl] * w
                return rc

            lax.fori_loop(0, K // LANES, group_body, 0)

        def slot(ci, q, b, idx_pf, gather_pf):
            gwait(q, b)
            scale(q, b)
            pltpu.sync_copy(rows.at[b], acc.at[dst_r.at[q]], add=True)
            if idx_pf:
                istart(ci + NIDX, q)
            if gather_pf:
                q2 = (q + 2) % NIDX
                iwait(ci + 2, q2)
                gstart(q2, b)

        for q in range(NIDX):
            istart(q, q)
        iwait(0, 0)
        gstart(0, 0)
        iwait(1, 1)
        gstart(1, 1)

        def mbody(mi, carry):
            for u in range(NIDX):
                slot(mi * NIDX + u, u, u % NROW, True, True)
            return carry

        lax.fori_loop(0, ch // NIDX - 1, mbody, 0)
        for u in range(NIDX):
            ci = (ch - NIDX) + u
            slot(ci, u, u % NROW, False, ci + 2 < ch)

        plsc.subcore_barrier()
        pltpu.sync_copy(acc.at[pl.ds(r0, rows_per_tile)],
                        out_hbm.at[c, pl.ds(r0, rows_per_tile)])

    return spmm(xw, srcm, dstm, ewm, zeros)


def kernel(X, edge_index, edge_weight, W, b):
    n_nodes, d_in = X.shape
    d_out = W.shape[1]
    n_edges = edge_weight.shape[0]
    xw = _matmul(X, W)
    src = edge_index[1].astype(jnp.int32)
    dst = edge_index[0].astype(jnp.int32)
    n_pad = ((n_nodes + NS * 8 - 1) // (NS * 8)) * (NS * 8)
    zeros = jnp.zeros((n_pad, d_out), jnp.float32)
    ch = -(-(-(-n_edges // NW) // K) // NIDX) * NIDX
    e_pad = NW * ch * K - n_edges
    srcm = jnp.concatenate(
        [src, jnp.zeros((e_pad,), jnp.int32)]).reshape(NW, ch, K)
    dstm = jnp.concatenate(
        [dst, jnp.zeros((e_pad,), jnp.int32)]).reshape(NW, ch, K)
    ewm = jnp.concatenate(
        [edge_weight.astype(jnp.float32),
         jnp.zeros((e_pad,), jnp.float32)]).reshape(NW, ch, K)
    partial = _sc_spmm(xw, srcm, dstm, ewm, zeros)
    return _combine(partial[:, :n_nodes], b)
